# baseline (speedup 1.0000x reference)
import functools

import jax
import jax.numpy as jnp
from jax import lax
from jax.experimental import pallas as pl
from jax.experimental.pallas import tpu as pltpu

EMBED_DIM = 64
NUM_EMB_ROWS = 144
BLK = 2048


def _tc_body(idx_ref, tab_ref, out_ref):
    idxb = idx_ref[0]
    iot = lax.broadcasted_iota(jnp.int32, (BLK, NUM_EMB_ROWS), 1)
    oh = jnp.where(iot == idxb, 1.0, 0.0).astype(jnp.float32)
    oh = oh.astype(jnp.bfloat16)
    out_ref[0] = jnp.dot(oh, tab_ref[...],
                         preferred_element_type=jnp.float32)


def _run_tc(idx_flat, table, interpret=False):
    n = idx_flat.shape[0]
    nb = n // BLK
    idx3 = idx_flat.reshape(nb, BLK, 1)
    out = pl.pallas_call(
        _tc_body,
        grid=(nb,),
        in_specs=[
            pl.BlockSpec((1, BLK, 1), lambda i: (i, 0, 0)),
            pl.BlockSpec((NUM_EMB_ROWS, EMBED_DIM), lambda i: (0, 0)),
        ],
        out_specs=pl.BlockSpec((1, BLK, EMBED_DIM), lambda i: (i, 0, 0)),
        out_shape=jax.ShapeDtypeStruct((nb, BLK, EMBED_DIM), jnp.float32),
        interpret=interpret,
    )(idx3, table.astype(jnp.bfloat16))
    return out.reshape(n, EMBED_DIM)


def kernel(channel_indices, table):
    b, f = channel_indices.shape
    idx_flat = channel_indices.reshape(b * f).astype(jnp.int32)
    out = _run_tc(idx_flat, table)
    return out.reshape(b, f, EMBED_DIM)


# speedup vs baseline: 1.3370x; 1.3370x over previous
import functools

import jax
import jax.numpy as jnp
from jax import lax
from jax.experimental import pallas as pl
from jax.experimental.pallas import tpu as pltpu

EMBED_DIM = 64
NUM_EMB_ROWS = 144
BLK = 2048


def _tc_body(idx_ref, tab_ref, out_ref):
    idxr = idx_ref[0]
    iot = lax.broadcasted_iota(jnp.int32, (NUM_EMB_ROWS, BLK), 0)
    oht = jnp.where(iot == idxr, 1.0, 0.0).astype(jnp.float32)
    out_ref[0] = lax.dot_general(
        oht, tab_ref[...],
        dimension_numbers=(((0,), (0,)), ((), ())),
        preferred_element_type=jnp.float32)


def _run_tc(idx_flat, table, interpret=False):
    n = idx_flat.shape[0]
    nb = n // BLK
    idx3 = idx_flat.reshape(nb, 1, BLK)
    out = pl.pallas_call(
        _tc_body,
        grid=(nb,),
        in_specs=[
            pl.BlockSpec((1, 1, BLK), lambda i: (i, 0, 0)),
            pl.BlockSpec((NUM_EMB_ROWS, EMBED_DIM), lambda i: (0, 0)),
        ],
        out_specs=pl.BlockSpec((1, BLK, EMBED_DIM), lambda i: (i, 0, 0)),
        out_shape=jax.ShapeDtypeStruct((nb, BLK, EMBED_DIM), jnp.float32),
        interpret=interpret,
    )(idx3, table)
    return out.reshape(n, EMBED_DIM)


def kernel(channel_indices, table):
    b, f = channel_indices.shape
    idx_flat = channel_indices.reshape(b * f).astype(jnp.int32)
    out = _run_tc(idx_flat, table)
    return out.reshape(b, f, EMBED_DIM)
